# baseline (speedup 1.0000x reference)
import jax
import jax.numpy as jnp
from jax import lax
from jax.experimental import pallas as pl
from jax.experimental.pallas import tpu as pltpu

M, N, DIM = 32, 32, 256
K = M * N
B = 256
T2_INV = 1.0 / (100.0 * 100.0)
KB = K // 2


def _som_body(x_ref, w_ref, wout_ref, loss_ref, dist_s, minv_s, mini_s):
    pid = pl.program_id(0)
    x = x_ref[...]
    w = w_ref[...]
    wout_ref[...] = w

    xw = lax.dot_general(
        x, w, (((1,), (1,)), ((), ())),
        preferred_element_type=jnp.float32,
    )
    w2 = lax.dot_general(
        jnp.ones((1, DIM), jnp.float32), w * w,
        (((1,), (1,)), ((), ())),
        preferred_element_type=jnp.float32,
    )
    x2 = jnp.sum(x * x, axis=1, keepdims=True)
    score = w2 - 2.0 * xw
    dist_s[:, pl.ds(pid * KB, KB)] = score + x2

    blkmin = jnp.min(score, axis=1, keepdims=True)
    kio = lax.broadcasted_iota(jnp.int32, (B, KB), 1) + pid * KB
    blkidx = jnp.min(jnp.where(score == blkmin, kio, K),
                     axis=1, keepdims=True)

    @pl.when(pid == 0)
    def _init():
        minv_s[...] = blkmin
        mini_s[...] = blkidx

    @pl.when(pid == 1)
    def _finish():
        better = blkmin < minv_s[...]
        bmu = jnp.where(better, blkidx, mini_s[...])

        bi = (bmu >> 5).astype(jnp.float32)
        bj = (bmu & 31).astype(jnp.float32)
        krow = lax.broadcasted_iota(jnp.int32, (1, K), 1)
        ki = (krow >> 5).astype(jnp.float32)
        kj = (krow & 31).astype(jnp.float32)

        man = jnp.abs(ki - bi) + jnp.abs(kj - bj)
        infl = jnp.exp(-(man * man) * T2_INV)
        rowsum = jnp.sum(dist_s[...] * infl, axis=1, keepdims=True)
        loss_ref[...] = jnp.sum(rowsum, axis=0, keepdims=True) * (1.0 / N)


def kernel(inputs, som_weights, locations):
    w_out, loss = pl.pallas_call(
        _som_body,
        grid=(2,),
        in_specs=[
            pl.BlockSpec((B, DIM), lambda i: (0, 0)),
            pl.BlockSpec((KB, DIM), lambda i: (i, 0)),
        ],
        out_specs=(
            pl.BlockSpec((KB, DIM), lambda i: (i, 0)),
            pl.BlockSpec((1, 1), lambda i: (0, 0)),
        ),
        out_shape=(
            jax.ShapeDtypeStruct((K, DIM), jnp.float32),
            jax.ShapeDtypeStruct((1, 1), jnp.float32),
        ),
        scratch_shapes=[
            pltpu.VMEM((B, K), jnp.float32),
            pltpu.VMEM((B, 1), jnp.float32),
            pltpu.VMEM((B, 1), jnp.int32),
        ],
    )(inputs, som_weights)
    return w_out, loss.reshape(())


# speedup vs baseline: 1.2009x; 1.2009x over previous
import jax
import jax.numpy as jnp
from jax import lax
from jax.experimental import pallas as pl

M, N, DIM = 32, 32, 256
K = M * N
B = 256
T2_INV = 1.0 / (100.0 * 100.0)


def _som_body(x_ref, w_ref, wout_ref, loss_ref):
    x = x_ref[...]
    w = w_ref[...]
    wout_ref[...] = w

    xw = lax.dot_general(
        x, w, (((1,), (1,)), ((), ())),
        preferred_element_type=jnp.float32,
    )
    w2 = lax.dot_general(
        jnp.ones((1, DIM), jnp.float32), w * w,
        (((1,), (1,)), ((), ())),
        preferred_element_type=jnp.float32,
    )
    x2 = jnp.sum(x * x, axis=1, keepdims=True)
    score = w2 - 2.0 * xw
    dist = score + x2

    bmu = jnp.argmin(score, axis=1).reshape(B, 1)

    bi = (bmu >> 5).astype(jnp.float32)
    bj = (bmu & 31).astype(jnp.float32)
    krow = lax.broadcasted_iota(jnp.int32, (1, K), 1)
    ki = (krow >> 5).astype(jnp.float32)
    kj = (krow & 31).astype(jnp.float32)

    man = jnp.abs(ki - bi) + jnp.abs(kj - bj)
    infl = jnp.exp(-(man * man) * T2_INV)
    rowsum = jnp.sum(dist * infl, axis=1, keepdims=True)
    loss_ref[...] = jnp.sum(rowsum, axis=0, keepdims=True) * (1.0 / N)


def kernel(inputs, som_weights, locations):
    w_out, loss = pl.pallas_call(
        _som_body,
        out_shape=(
            jax.ShapeDtypeStruct((K, DIM), jnp.float32),
            jax.ShapeDtypeStruct((1, 1), jnp.float32),
        ),
    )(inputs, som_weights)
    return w_out, loss.reshape(())


# speedup vs baseline: 1.2091x; 1.0068x over previous
import jax
import jax.numpy as jnp
from jax import lax
from jax.experimental import pallas as pl

M, N, DIM = 32, 32, 256
K = M * N
B = 256
T2_INV = 1.0 / (100.0 * 100.0)


def _som_body(x_ref, w_ref, wout_ref, loss_ref):
    x = x_ref[...]
    w = w_ref[...]
    wout_ref[...] = w

    xwn = lax.dot_general(
        -2.0 * x, w, (((1,), (1,)), ((), ())),
        preferred_element_type=jnp.float32,
    )
    w2 = lax.dot_general(
        jnp.ones((1, DIM), jnp.float32), w * w,
        (((1,), (1,)), ((), ())),
        preferred_element_type=jnp.float32,
    )
    x2 = jnp.sum(x * x, axis=1, keepdims=True)
    score = w2 + xwn
    dist = score + x2

    bmu = jnp.argmin(score, axis=1).reshape(B, 1)

    bi = (bmu >> 5).astype(jnp.float32)
    bj = (bmu & 31).astype(jnp.float32)
    krow = lax.broadcasted_iota(jnp.int32, (1, K), 1)
    ki = (krow >> 5).astype(jnp.float32)
    kj = (krow & 31).astype(jnp.float32)

    man = jnp.abs(ki - bi) + jnp.abs(kj - bj)
    infl = jnp.exp(-(man * man) * T2_INV)
    rowsum = jnp.sum(dist * infl, axis=1, keepdims=True)
    loss_ref[...] = jnp.sum(rowsum, axis=0, keepdims=True) * (1.0 / N)


def kernel(inputs, som_weights, locations):
    w_out, loss = pl.pallas_call(
        _som_body,
        out_shape=(
            jax.ShapeDtypeStruct((K, DIM), jnp.float32),
            jax.ShapeDtypeStruct((1, 1), jnp.float32),
        ),
    )(inputs, som_weights)
    return w_out, loss.reshape(())
